# Initial kernel scaffold; baseline (speedup 1.0000x reference)
import jax
import jax.numpy as jnp
from jax.experimental import pallas as pl


def kernel(scores, k):
    raise NotImplementedError("write your pallas kernel here")


# speedup vs baseline: 4.4712x; 4.4712x over previous
import jax
import jax.numpy as jnp
from jax.experimental import pallas as pl
from jax.experimental.pallas import tpu as pltpu

def _body(k_ref, x_ref, o_ref):
    int_min = jnp.int32(-(2**31))
    x = x_ref[...]
    k = k_ref[0]
    ibits = jax.lax.bitcast_convert_type(x, jnp.int32)
    skey = jnp.where(ibits >= 0, ibits, ibits ^ jnp.int32(0x7FFFFFFF))

    rows = x.shape[0]

    def step(b, prefix_u):
        bit = jnp.int32(1) << (31 - b)
        cand_u = prefix_u | bit
        cand_s = cand_u ^ int_min
        cnt = jnp.sum((skey >= cand_s).astype(jnp.int32), axis=1,
                      keepdims=True)
        return jnp.where(cnt >= k, cand_u, prefix_u)

    prefix_u = jax.lax.fori_loop(0, 32, step,
                                 jnp.zeros((rows, 1), jnp.int32))
    thresh_s = prefix_u ^ int_min

    mask = skey >= thresh_s
    m = jnp.max(x, axis=1, keepdims=True)
    e = jnp.where(mask, jnp.exp(x - m), 0.0)
    z = jnp.sum(e, axis=1, keepdims=True)
    o_ref[...] = e / z


def kernel(scores, k):
    rows, n = scores.shape
    r_blk = 8
    k_arr = jnp.reshape(jnp.asarray(k, jnp.int32), (1,))
    return pl.pallas_call(
        _body,
        grid=(rows // r_blk,),
        in_specs=[
            pl.BlockSpec(memory_space=pltpu.SMEM),
            pl.BlockSpec((r_blk, n), lambda i: (i, 0)),
        ],
        out_specs=pl.BlockSpec((r_blk, n), lambda i: (i, 0)),
        out_shape=jax.ShapeDtypeStruct(scores.shape, scores.dtype),
    )(k_arr, scores)
